# Initial kernel scaffold; baseline (speedup 1.0000x reference)
import jax
import jax.numpy as jnp
from jax.experimental import pallas as pl


def kernel(node_indices, time_delta_ms, path_length, temporal_table, path_table, lap_pe, lap_W, lap_b):
    raise NotImplementedError("write your pallas kernel here")


# speedup vs baseline: 10.3884x; 10.3884x over previous
import functools

import jax
import jax.numpy as jnp
from jax import lax
from jax.experimental import pallas as pl
from jax.experimental.pallas import tpu as pltpu
from jax.experimental.pallas import tpu_sc as plsc

_IDX_PER_DMA = 128
_NUM_WORKERS = 32
_CHUNK_ROWS = 1024
_DMAS_PER_CHUNK = _CHUNK_ROWS // _IDX_PER_DMA


def _sc_gather_body(idx_hbm, table_hbm, out_hbm, idx_v, rows_v, sem_g, n_chunks, rows_per_worker):
    wid = lax.axis_index("s") * 2 + lax.axis_index("c")
    base = wid * rows_per_worker
    ibase = wid * (rows_per_worker // _IDX_PER_DMA)

    def start_chunk(c, buf):
        pltpu.sync_copy(
            idx_hbm.at[pl.ds(ibase + c * _DMAS_PER_CHUNK, _DMAS_PER_CHUNK)],
            idx_v.at[buf],
        )
        for j in range(_DMAS_PER_CHUNK):
            pltpu.async_copy(
                table_hbm.at[idx_v.at[buf, j]],
                rows_v.at[buf, pl.ds(j * _IDX_PER_DMA, _IDX_PER_DMA)],
                sem_g,
            )

    def drain_chunk(c, buf):
        for j in range(_DMAS_PER_CHUNK):
            pltpu.make_async_copy(
                table_hbm.at[idx_v.at[buf, j]],
                rows_v.at[buf, pl.ds(j * _IDX_PER_DMA, _IDX_PER_DMA)],
                sem_g,
            ).wait()
        pltpu.sync_copy(
            rows_v.at[buf],
            out_hbm.at[pl.ds(base + c * _CHUNK_ROWS, _CHUNK_ROWS)],
        )

    start_chunk(0, 0)

    def loop_body(c, carry):
        buf = lax.rem(c, 2)
        nxt = lax.rem(c + 1, 2)

        @pl.when(c + 1 < n_chunks)
        def _():
            start_chunk(c + 1, nxt)

        drain_chunk(c, buf)
        return carry

    lax.fori_loop(0, n_chunks, loop_body, 0, unroll=False)


def _sc_gather(idx, table):
    n = idx.shape[0] * idx.shape[1]
    k = table.shape[1]
    rows_per_worker = n // _NUM_WORKERS
    n_chunks = rows_per_worker // _CHUNK_ROWS
    mesh = plsc.VectorSubcoreMesh(core_axis_name="c", subcore_axis_name="s")
    body = functools.partial(
        _sc_gather_body, n_chunks=n_chunks, rows_per_worker=rows_per_worker
    )
    f = pl.kernel(
        body,
        out_type=jax.ShapeDtypeStruct((n, k), jnp.float32),
        mesh=mesh,
        scratch_types=[
            pltpu.VMEM((2, _DMAS_PER_CHUNK, _IDX_PER_DMA), jnp.int32),
            pltpu.VMEM((2, _CHUNK_ROWS, k), jnp.float32),
            pltpu.SemaphoreType.DMA,
        ],
        compiler_params=pltpu.CompilerParams(use_tc_tiling_on_sc=False),
        name="hybrid_pe_sc_gather",
    )
    return f(idx, table)


_BLK = 4096


def _tc_combine_body(feats_ref, td_ref, pl_ref, tt_ref, pt_ref, w_ref, b_ref, out_ref, *, n_t, n_p):
    feats = feats_ref[...]
    acc = jnp.dot(feats, w_ref[...], preferred_element_type=jnp.float32)
    acc += b_ref[...]

    td = td_ref[...]
    tb = jnp.floor(jnp.log(td + 1.0) * 0.43429448190325176)
    tb = jnp.clip(tb, 0.0, float(n_t - 1)).astype(jnp.int32)
    tt = tt_ref[...]
    for j in range(n_t):
        acc += jnp.where(tb == j, 1.0, 0.0) * tt[j : j + 1, :]

    pb = jnp.clip(pl_ref[...], 0, n_p - 1)
    pt = pt_ref[...]
    for j in range(n_p):
        acc += jnp.where(pb == j, 1.0, 0.0) * pt[j : j + 1, :]

    out_ref[...] = acc


def _tc_combine(feats, td, pathlen, t_table, p_table, w, b):
    n, k = feats.shape
    d = w.shape[1]
    n_t = t_table.shape[0]
    n_p = p_table.shape[0]
    grid = (n // _BLK,)
    body = functools.partial(_tc_combine_body, n_t=n_t, n_p=n_p)
    return pl.pallas_call(
        body,
        grid=grid,
        in_specs=[
            pl.BlockSpec((_BLK, k), lambda i: (i, 0)),
            pl.BlockSpec((_BLK, 1), lambda i: (i, 0)),
            pl.BlockSpec((_BLK, 1), lambda i: (i, 0)),
            pl.BlockSpec((n_t, d), lambda i: (0, 0)),
            pl.BlockSpec((n_p, d), lambda i: (0, 0)),
            pl.BlockSpec((k, d), lambda i: (0, 0)),
            pl.BlockSpec((1, d), lambda i: (0, 0)),
        ],
        out_specs=pl.BlockSpec((_BLK, d), lambda i: (i, 0)),
        out_shape=jax.ShapeDtypeStruct((n, d), jnp.float32),
        name="hybrid_pe_tc_combine",
    )(feats, td, pathlen, t_table, p_table, w, b)


def kernel(node_indices, time_delta_ms, path_length, temporal_table, path_table, lap_pe, lap_W, lap_b):
    batch, seq = node_indices.shape
    n = batch * seq
    d = lap_W.shape[1]

    idx = node_indices.reshape(n // _IDX_PER_DMA, _IDX_PER_DMA).astype(jnp.int32)
    feats = _sc_gather(idx, lap_pe)

    td = time_delta_ms.reshape(n, 1)
    pathlen = path_length.reshape(n, 1).astype(jnp.int32)
    out = _tc_combine(
        feats, td, pathlen, temporal_table, path_table, lap_W,
        lap_b.reshape(1, d),
    )
    return out.reshape(batch, seq, d)
